# baseline (speedup 1.0000x reference)
import functools

import jax
import jax.numpy as jnp
from jax import lax
from jax.experimental import pallas as pl
from jax.experimental.pallas import tpu as pltpu
from jax.experimental.pallas import tpu_sc as plsc

GAMMA = 3.0
N = 625
NPAD = 640
B = 16
NCHUNK = NPAD // 16
NROWS = 14


def _sc_task_body(x_hbm, out_hbm, xv, kv, uv, vv, sv):
    cid = lax.axis_index("c")
    sid = lax.axis_index("s")

    pltpu.sync_copy(x_hbm.at[sid], xv)

    ones = jnp.full((16,), 1.0, jnp.float32)
    zeros = jnp.zeros((16,), jnp.float32)
    w1 = jnp.full((16,), 1.0 - cid.astype(jnp.float32), jnp.float32)
    w2 = ones - w1

    pacc = zeros
    for c in range(NCHUNK):
        ds = pl.ds(c * 16, 16)
        cls1 = xv[0, ds]
        labf = xv[1, ds]
        mf = jnp.where(labf > 0.5, ones, zeros)
        pp = jnp.exp(cls1)

        sh0 = xv[10, ds]
        sh1 = xv[11, ds]
        sh2 = xv[12, ds]
        sh3 = xv[13, ds]

        def corners(base):
            cx = xv[base + 0, ds] * sh2 + sh0
            cy = xv[base + 1, ds] * sh3 + sh1
            w = jnp.exp(xv[base + 2, ds]) * sh2
            h = jnp.exp(xv[base + 3, ds]) * sh3
            hw = w * 0.5
            hh = h * 0.5
            return cx - hw, cy - hh, cx + hw, cy + hh

        ax1, ay1, ax2, ay2 = corners(2)
        bx1, by1, bx2, by2 = corners(6)

        ix1 = jnp.maximum(ax1, bx1)
        iy1 = jnp.maximum(ay1, by1)
        ix2 = jnp.minimum(ax2, bx2)
        iy2 = jnp.minimum(ay2, by2)
        inter = jnp.maximum(ix2 - ix1, 0.0) * jnp.maximum(iy2 - iy1, 0.0)
        area_a = jnp.maximum(ax2 - ax1, 0.0) * jnp.maximum(ay2 - ay1, 0.0)
        area_b = jnp.maximum(bx2 - bx1, 0.0) * jnp.maximum(by2 - by1, 0.0)
        iou = inter / jnp.maximum(area_a + area_b - inter, 1e-6)

        key = w1 * iou + w2 * pp
        val = w1 * pp + w2 * iou
        kv[ds] = key
        uv[ds] = mf * jnp.exp(-GAMMA * val)
        vv[ds] = mf * jnp.exp(GAMMA * val)
        pacc = pacc + mf

    iotav = lax.broadcasted_iota(jnp.int32, (16,), 0)
    gdn = lax.GatherDimensionNumbers(
        offset_dims=(), collapsed_slice_dims=(0,), start_index_map=(0,))

    def lane_bcast(vec, j):
        idx = jnp.full((16,), j, jnp.int32)
        return lax.gather(vec, idx[:, None], gdn, (1,),
                          mode=lax.GatherScatterMode.PROMISE_IN_BOUNDS)

    def body(b, sacc):
        bvec = jnp.full((16,), b, jnp.int32)
        start = (b // 16) * 16
        j = b - start
        kb = lane_bcast(kv[pl.ds(start, 16)], j)
        vb = lane_bcast(vv[pl.ds(start, 16)], j)
        acc = zeros
        for c in range(NCHUNK):
            ds = pl.ds(c * 16, 16)
            ka = kv[ds]
            ua = uv[ds]
            ia = iotav + (c * 16)
            pred = (ka > kb) | ((ka == kb) & (ia < bvec))
            acc = acc + jnp.where(pred, ua, zeros)
        return sacc + acc * vb

    sacc = lax.fori_loop(0, NPAD, body, zeros)

    sv[0, :] = sacc
    sv[1, :] = pacc
    pltpu.sync_copy(sv, out_hbm.at[sid * 2 + cid])


def _sc_call(x):
    mesh = plsc.VectorSubcoreMesh(core_axis_name="c", subcore_axis_name="s")
    k = functools.partial(
        pl.kernel,
        mesh=mesh,
        out_type=jax.ShapeDtypeStruct((2 * B, 2, 16), jnp.float32),
        scratch_types=[
            pltpu.VMEM((NROWS, NPAD), jnp.float32),
            pltpu.VMEM((NPAD,), jnp.float32),
            pltpu.VMEM((NPAD,), jnp.float32),
            pltpu.VMEM((NPAD,), jnp.float32),
            pltpu.VMEM((2, 16), jnp.float32),
        ],
    )(_sc_task_body)
    return k(x)


def _finalize_kernel(x_ref, f1_ref, f2_ref):
    x = x_ref[...]
    s = jnp.sum(x[:, 0, :], axis=1, keepdims=True)
    p = jnp.sum(x[:, 1, :], axis=1, keepdims=True)
    rowid = lax.broadcasted_iota(jnp.int32, (2 * B, 1), 0)
    is1 = (rowid % 2) == 0
    npairs = jnp.maximum(p * (p - 1.0) * 0.5, 1.0)
    include = (p >= 2.0).astype(jnp.float32)
    contrib = include * s / npairs
    total1 = jnp.sum(jnp.where(is1, contrib, 0.0))
    total2 = jnp.sum(jnp.where(is1, 0.0, contrib))
    count = jnp.sum(jnp.where(is1, include, 0.0))
    denom = jnp.maximum(count, 1.0)
    has = (count > 0.0).astype(jnp.float32)
    f1_ref[...] = (total1 / denom * has).reshape(1, 1)
    f2_ref[...] = (total2 / denom * has).reshape(1, 1)


def kernel(cls, label_cls, pred_loc, label_loc, shape):
    pad = NPAD - N
    cls1 = jnp.pad(cls.reshape(B, N, 2)[:, :, 1], ((0, 0), (0, pad)))
    labf = jnp.pad(label_cls.reshape(B, N).astype(jnp.float32),
                   ((0, 0), (0, pad)))
    ploc = jnp.pad(pred_loc.reshape(B, 4, N), ((0, 0), (0, 0), (0, pad)))
    lloc = jnp.pad(label_loc.reshape(B, 4, N), ((0, 0), (0, 0), (0, pad)))
    shp = jnp.pad(shape.reshape(4, N), ((0, 0), (0, pad)),
                  constant_values=1.0)
    shp_b = jnp.broadcast_to(shp[None], (B, 4, NPAD))
    x = jnp.concatenate(
        [cls1[:, None, :], labf[:, None, :], ploc, lloc, shp_b], axis=1)

    parts = _sc_call(x)

    f1, f2 = pl.pallas_call(
        _finalize_kernel,
        out_shape=[
            jax.ShapeDtypeStruct((1, 1), jnp.float32),
            jax.ShapeDtypeStruct((1, 1), jnp.float32),
        ],
    )(parts)
    return (f1.reshape(()), f2.reshape(()))
